# Initial kernel scaffold; baseline (speedup 1.0000x reference)
import jax
import jax.numpy as jnp
from jax.experimental import pallas as pl


def kernel(x, edge_index, edge_attr, line_graph_edge_index, W_proj, W1, b1, prelu_a, bn_gamma, bn_beta):
    raise NotImplementedError("write your pallas kernel here")


# speedup vs baseline: 14.4176x; 14.4176x over previous
import functools

import jax
import jax.numpy as jnp
from jax import lax
from jax.experimental import pallas as pl
from jax.experimental.pallas import tpu as pltpu
from jax.experimental.pallas import tpu_sc as plsc

N_NODES = 10000
N_EDGES = 320000
L_EDGES = 1280000
NODE_DIM = 128
EDGE_DIM = 16

NC = 2
NS = 16
NW = NC * NS

NP = 3
SEG_R = 106672
SEG_PAD = 106752
TILE_SEG = SEG_PAD // NS
OUT_SEG = NP * SEG_R + (SEG_PAD - SEG_R)
L_PER_TILE = L_EDGES // NW
L_CHUNK = 2000
G = 128
CLIST = L_CHUNK + G
ZROWS = 128

N_PAD = 10240
NTILE_SEG = N_PAD // NS
E_PER_TILE = N_EDGES // NW
E_CHUNK = 2000
GN = 80
G2 = 80

@functools.cache
def _mesh():
    return plsc.VectorSubcoreMesh(
        core_axis_name="c", subcore_axis_name="s",
        num_cores=NC, num_subcores=NS)


def _proj_body(x_ref, wt_ref, out_ref):
    out_ref[...] = 0.5 * jnp.dot(
        x_ref[...], wt_ref[...], preferred_element_type=jnp.float32)


def _proj(x, w_t):
    return pl.pallas_call(
        _proj_body,
        out_shape=jax.ShapeDtypeStruct((N_NODES, EDGE_DIM), jnp.float32),
    )(x, w_t)


def _fused_body(xp_hbm, src_hbm, dst_hbm, ea_hbm, out_hbm,
                idx_s, idx_d, rows_s, rows_d, acc, sem_s, sem_d):
    c = lax.axis_index("c")
    s = lax.axis_index("s")
    wid = c * NS + s
    base = wid * E_PER_TILE
    nchunks = E_PER_TILE // E_CHUNK

    def chunk(k, _):
        off = base + k * E_CHUNK
        pltpu.sync_copy(src_hbm.at[pl.ds(off, E_CHUNK)], idx_s)
        pltpu.sync_copy(dst_hbm.at[pl.ds(off, E_CHUNK)], idx_d)
        pltpu.sync_copy(ea_hbm.at[pl.ds(off, E_CHUNK)], acc)

        def batch(b, _):
            boff = pl.multiple_of(b * G2, 8)
            cp_s = pltpu.async_copy(
                xp_hbm.at[idx_s.at[pl.ds(boff, G2)]], rows_s, sem_s)
            cp_d = pltpu.async_copy(
                xp_hbm.at[idx_d.at[pl.ds(boff, G2)]], rows_d, sem_d)
            cp_s.wait()
            cp_d.wait()

            def row(r, _):
                acc[boff + r] = acc[boff + r] + rows_s[r] + rows_d[r]
                return 0

            lax.fori_loop(0, G2, row, 0)
            return 0

        lax.fori_loop(0, E_CHUNK // G2, batch, 0)
        pltpu.sync_copy(acc, out_hbm.at[pl.ds(off, E_CHUNK)])
        return 0

    lax.fori_loop(0, nchunks, chunk, 0)


def _fused_stage(xp_half, src, dst, edge_attr):
    k = pl.kernel(
        _fused_body,
        out_type=jax.ShapeDtypeStruct((N_EDGES, EDGE_DIM), jnp.float32),
        mesh=_mesh(),
        compiler_params=pltpu.CompilerParams(use_tc_tiling_on_sc=False, needs_layout_passes=False),
        scratch_types=[
            pltpu.VMEM((E_CHUNK,), jnp.int32),
            pltpu.VMEM((E_CHUNK,), jnp.int32),
            pltpu.VMEM((G2, EDGE_DIM), jnp.float32),
            pltpu.VMEM((G2, EDGE_DIM), jnp.float32),
            pltpu.VMEM((E_CHUNK, EDGE_DIM), jnp.float32),
            pltpu.SemaphoreType.DMA,
            pltpu.SemaphoreType.DMA,
        ],
    )
    return k(xp_half, src, dst, edge_attr)


def _line_body(fused_hbm, lsrc_hbm, ldst_hbm, sums_hbm, cnts_hbm,
               lsrc, ldst, clist, dlist, dbuf, rows, ones, zb, zc,
               sums_sp, cnts_sp, gsem):
    c = lax.axis_index("c")
    s = lax.axis_index("s")
    wid = c * NS + s
    ibase = wid * L_PER_TILE

    one_v = jnp.ones((16,), jnp.float32)
    zero_v = jnp.zeros((16,), jnp.float32)
    for j in range(G // 16):
        ones[pl.ds(j * 16, 16)] = one_v
    for j in range(ZROWS // 16):
        zc[pl.ds(j * 16, 16)] = zero_v

    def zrow(r, _):
        zb[r] = zero_v
        return 0

    lax.fori_loop(0, ZROWS, zrow, 0)

    for p in range(NP):
        seg_base = p * SEG_R
        row0 = s * TILE_SEG
        nfull = TILE_SEG // ZROWS
        for j in range(nfull):
            pltpu.sync_copy(zb, sums_sp.at[pl.ds(row0 + j * ZROWS, ZROWS)])
            pltpu.sync_copy(zc, cnts_sp.at[pl.ds(row0 + j * ZROWS, ZROWS)])
        rem = TILE_SEG - nfull * ZROWS
        if rem:
            pltpu.sync_copy(zb.at[pl.ds(0, rem)],
                            sums_sp.at[pl.ds(row0 + nfull * ZROWS, rem)])
            pltpu.sync_copy(zc.at[pl.ds(0, rem)],
                            cnts_sp.at[pl.ds(row0 + nfull * ZROWS, rem)])
        plsc.subcore_barrier()

        zi = jnp.zeros((16,), jnp.int32)
        di = jnp.full((16,), SEG_R, jnp.int32)

        def chunk(k, _):
            off = ibase + k * L_CHUNK
            pltpu.sync_copy(lsrc_hbm.at[pl.ds(off, L_CHUNK)], lsrc)
            pltpu.sync_copy(ldst_hbm.at[pl.ds(off, L_CHUNK)], ldst)

            def vec(i, cur):
                d = ldst[pl.ds(i * 16, 16)]
                loc = d - seg_base
                mask = (loc >= 0) & (loc < SEG_R)
                loc = jnp.where(mask, loc, SEG_R)
                u = lsrc[pl.ds(i * 16, 16)]
                plsc.store_compressed(clist.at[pl.ds(cur, 16)], u, mask=mask)
                plsc.store_compressed(dlist.at[pl.ds(cur, 16)], loc, mask=mask)
                return cur + jnp.sum(mask.astype(jnp.int32))

            nc = lax.fori_loop(0, L_CHUNK // 16, vec, 0)

            for j in range(G // 16):
                clist[pl.ds(nc + j * 16, 16)] = zi
                dlist[pl.ds(nc + j * 16, 16)] = di

            def batch(b, _):
                boff = pl.multiple_of(b * G, 8)
                cp = pltpu.async_copy(
                    fused_hbm.at[clist.at[pl.ds(boff, G)]], rows, gsem)
                for j in range(G // 16):
                    dbuf[pl.ds(j * 16, 16)] = dlist[pl.ds(boff + j * 16, 16)]
                cp.wait()
                pltpu.sync_copy(rows, sums_sp.at[dbuf], add=True)
                pltpu.sync_copy(ones, cnts_sp.at[dbuf], add=True)
                return 0

            nb = (nc + G - 1) // G
            lax.fori_loop(0, nb, batch, 0)
            return 0

        lax.fori_loop(0, L_PER_TILE // L_CHUNK, chunk, 0)
        plsc.subcore_barrier()

        pltpu.sync_copy(
            sums_sp.at[pl.ds(row0, TILE_SEG)],
            sums_hbm.at[c, pl.ds(seg_base + row0, TILE_SEG)])
        pltpu.sync_copy(
            cnts_sp.at[pl.ds(row0, TILE_SEG)],
            cnts_hbm.at[c, pl.ds(seg_base + row0, TILE_SEG)])
        plsc.subcore_barrier()


def _line_stage(fused, lsrc, ldst):
    k = pl.kernel(
        _line_body,
        out_type=(
            jax.ShapeDtypeStruct((NC, OUT_SEG, EDGE_DIM), jnp.float32),
            jax.ShapeDtypeStruct((NC, OUT_SEG), jnp.float32),
        ),
        mesh=_mesh(),
        compiler_params=pltpu.CompilerParams(use_tc_tiling_on_sc=False, needs_layout_passes=False),
        scratch_types=[
            pltpu.VMEM((L_CHUNK,), jnp.int32),
            pltpu.VMEM((L_CHUNK,), jnp.int32),
            pltpu.VMEM((CLIST,), jnp.int32),
            pltpu.VMEM((CLIST,), jnp.int32),
            pltpu.VMEM((G,), jnp.int32),
            pltpu.VMEM((G, EDGE_DIM), jnp.float32),
            pltpu.VMEM((G,), jnp.float32),
            pltpu.VMEM((ZROWS, EDGE_DIM), jnp.float32),
            pltpu.VMEM((ZROWS,), jnp.float32),
            pltpu.VMEM_SHARED((SEG_PAD, EDGE_DIM), jnp.float32),
            pltpu.VMEM_SHARED((SEG_PAD,), jnp.float32),
            pltpu.SemaphoreType.DMA,
        ],
    )
    return k(fused, lsrc, ldst)


BLK = 8000
NBLK = N_EDGES // BLK


def _mlp_body(s0_ref, s1_ref, c0_ref, c1_ref, w1t_ref, b1_ref, a_ref,
              pre_ref, stats_ref, acc_ref):
    i = pl.program_id(0)
    ssum = s0_ref[0] + s1_ref[0]
    cnt = jnp.maximum(c0_ref[0] + c1_ref[0], 1.0)
    agg = ssum / cnt
    h = jnp.dot(agg, w1t_ref[...], preferred_element_type=jnp.float32)
    h = h + b1_ref[...]
    h = jnp.where(h >= 0.0, h, a_ref[0, 0] * h)
    pre_ref[...] = h

    @pl.when(i == 0)
    def _():
        acc_ref[...] = jnp.zeros_like(acc_ref)

    part = jnp.concatenate(
        [jnp.sum(h, axis=0, keepdims=True),
         jnp.sum(h * h, axis=0, keepdims=True)], axis=0)
    acc_ref[0:2, :] = acc_ref[0:2, :] + part

    @pl.when(i == NBLK - 1)
    def _():
        stats_ref[...] = acc_ref[...]


def _mlp_stage(sums, cnts, w1t, b1r, ar):
    return pl.pallas_call(
        _mlp_body,
        grid=(NBLK,),
        in_specs=[
            pl.BlockSpec((1, BLK, EDGE_DIM), lambda i: (0, i, 0)),
            pl.BlockSpec((1, BLK, EDGE_DIM), lambda i: (1, i, 0)),
            pl.BlockSpec((1, BLK, 1), lambda i: (0, i, 0)),
            pl.BlockSpec((1, BLK, 1), lambda i: (1, i, 0)),
            pl.BlockSpec((EDGE_DIM, EDGE_DIM), lambda i: (0, 0)),
            pl.BlockSpec((1, EDGE_DIM), lambda i: (0, 0)),
            pl.BlockSpec((1, 1), lambda i: (0, 0)),
        ],
        out_specs=[
            pl.BlockSpec((BLK, EDGE_DIM), lambda i: (i, 0)),
            pl.BlockSpec((8, EDGE_DIM), lambda i: (0, 0)),
        ],
        out_shape=[
            jax.ShapeDtypeStruct((N_EDGES, EDGE_DIM), jnp.float32),
            jax.ShapeDtypeStruct((8, EDGE_DIM), jnp.float32),
        ],
        scratch_shapes=[pltpu.VMEM((8, EDGE_DIM), jnp.float32)],
    )(sums, sums, cnts, cnts, w1t, b1r, ar)


def _bn_body(pre_ref, fused_ref, stats_ref, g_ref, be_ref, out_ref):
    n = float(N_EDGES)
    mu = stats_ref[0:1, :] / n
    var = stats_ref[1:2, :] / n - mu * mu
    inv = lax.rsqrt(var + 1e-5)
    out_ref[...] = fused_ref[...] + (
        (pre_ref[...] - mu) * inv * g_ref[...] + be_ref[...])


def _bn_stage(pre, fused, stats, gr, ber):
    return pl.pallas_call(
        _bn_body,
        grid=(NBLK,),
        in_specs=[
            pl.BlockSpec((BLK, EDGE_DIM), lambda i: (i, 0)),
            pl.BlockSpec((BLK, EDGE_DIM), lambda i: (i, 0)),
            pl.BlockSpec((8, EDGE_DIM), lambda i: (0, 0)),
            pl.BlockSpec((1, EDGE_DIM), lambda i: (0, 0)),
            pl.BlockSpec((1, EDGE_DIM), lambda i: (0, 0)),
        ],
        out_specs=pl.BlockSpec((BLK, EDGE_DIM), lambda i: (i, 0)),
        out_shape=jax.ShapeDtypeStruct((N_EDGES, EDGE_DIM), jnp.float32),
    )(pre, fused, stats, gr, ber)


def _node_body(f2_hbm, dst_hbm, sums_hbm, cnts_hbm,
               didx, rows, dbuf, ones, zb, zc, sums_sp, cnts_sp):
    c = lax.axis_index("c")
    s = lax.axis_index("s")
    wid = c * NS + s
    base = wid * E_PER_TILE

    one_v = jnp.ones((16,), jnp.float32)
    zero_v = jnp.zeros((16,), jnp.float32)
    for j in range(GN // 16):
        ones[pl.ds(j * 16, 16)] = one_v
    for j in range(1024 // 16):
        zc[pl.ds(j * 16, 16)] = zero_v

    def zrow(r, _):
        zb[r] = zero_v
        return 0
    lax.fori_loop(0, 1024, zrow, 0)

    row0 = s * NTILE_SEG
    pltpu.sync_copy(zb.at[pl.ds(0, NTILE_SEG)],
                    sums_sp.at[pl.ds(row0, NTILE_SEG)])
    pltpu.sync_copy(zc.at[pl.ds(0, NTILE_SEG)],
                    cnts_sp.at[pl.ds(row0, NTILE_SEG)])
    plsc.subcore_barrier()

    nchunks = E_PER_TILE // E_CHUNK

    def chunk(k, _):
        off = base + k * E_CHUNK
        pltpu.sync_copy(dst_hbm.at[pl.ds(off, E_CHUNK)], didx)
        pltpu.sync_copy(f2_hbm.at[pl.ds(off, E_CHUNK)], rows)

        def batch(b, _):
            boff = pl.multiple_of(b * GN, 8)
            for j in range(GN // 16):
                dbuf[pl.ds(j * 16, 16)] = didx[pl.ds(boff + j * 16, 16)]
            pltpu.sync_copy(rows.at[pl.ds(boff, GN)],
                            sums_sp.at[dbuf], add=True)
            pltpu.sync_copy(ones, cnts_sp.at[dbuf], add=True)
            return 0

        lax.fori_loop(0, E_CHUNK // GN, batch, 0)
        return 0

    lax.fori_loop(0, nchunks, chunk, 0)
    plsc.subcore_barrier()

    pltpu.sync_copy(sums_sp.at[pl.ds(row0, NTILE_SEG)],
                    sums_hbm.at[c, pl.ds(row0, NTILE_SEG)])
    pltpu.sync_copy(cnts_sp.at[pl.ds(row0, NTILE_SEG)],
                    cnts_hbm.at[c, pl.ds(row0, NTILE_SEG)])


def _node_stage(fused2, dst):
    k = pl.kernel(
        _node_body,
        out_type=(
            jax.ShapeDtypeStruct((NC, N_PAD, EDGE_DIM), jnp.float32),
            jax.ShapeDtypeStruct((NC, N_PAD), jnp.float32),
        ),
        mesh=_mesh(),
        compiler_params=pltpu.CompilerParams(use_tc_tiling_on_sc=False, needs_layout_passes=False),
        scratch_types=[
            pltpu.VMEM((E_CHUNK,), jnp.int32),
            pltpu.VMEM((E_CHUNK, EDGE_DIM), jnp.float32),
            pltpu.VMEM((GN,), jnp.int32),
            pltpu.VMEM((GN,), jnp.float32),
            pltpu.VMEM((1024, EDGE_DIM), jnp.float32),
            pltpu.VMEM((1024,), jnp.float32),
            pltpu.VMEM_SHARED((N_PAD, EDGE_DIM), jnp.float32),
            pltpu.VMEM_SHARED((N_PAD,), jnp.float32),
        ],
    )
    return k(fused2, dst)


def _comb_body(s_ref, c_ref, out_ref):
    cnt = jnp.maximum(c_ref[0] + c_ref[1], 1.0)
    out_ref[...] = (s_ref[0] + s_ref[1]) / cnt


def _comb_stage(nsums, ncnts):
    return pl.pallas_call(
        _comb_body,
        out_shape=jax.ShapeDtypeStruct((N_PAD, EDGE_DIM), jnp.float32),
    )(nsums, ncnts)


def kernel(x, edge_index, edge_attr, line_graph_edge_index,
           W_proj, W1, b1, prelu_a, bn_gamma, bn_beta):
    src = edge_index[0]
    dst = edge_index[1]
    xp_half = _proj(x, W_proj.T)
    fused = _fused_stage(xp_half, src, dst, edge_attr)
    sums, cnts = _line_stage(
        fused, line_graph_edge_index[0], line_graph_edge_index[1])
    pre, stats = _mlp_stage(
        sums[:, :N_EDGES], cnts[:, :N_EDGES].reshape(NC, N_EDGES, 1),
        W1.T, b1.reshape(1, EDGE_DIM), prelu_a.reshape(1, 1))
    fused2 = _bn_stage(pre, fused, stats,
                       bn_gamma.reshape(1, EDGE_DIM),
                       bn_beta.reshape(1, EDGE_DIM))
    nsums, ncnts = _node_stage(fused2, dst)
    out = _comb_stage(nsums, ncnts.reshape(NC, N_PAD, 1))
    return out[:N_NODES]


# speedup vs baseline: 28.0631x; 1.9464x over previous
import functools

import jax
import jax.numpy as jnp
from jax import lax
from jax.experimental import pallas as pl
from jax.experimental.pallas import tpu as pltpu
from jax.experimental.pallas import tpu_sc as plsc

N_NODES = 10000
N_EDGES = 320000
L_EDGES = 1280000
NODE_DIM = 128
EDGE_DIM = 16

NC = 2
NS = 16
NW = NC * NS

G = 128
ZROWS = 128
DCH = 512

NP = 2
SEG_S = 80128
SEG_SP = SEG_S + 16
TILE_SEG = SEG_S // NS
AGG_ROWS = NC * NP * SEG_S
L_PER_TILE = L_EDGES // NS
L_CHUNK = 4000
CLIST = L_CHUNK + 2 * G

N_S = 5120
N_SP = N_S + 16
NTILE_SEG = N_S // NS
E_PER_TILE = N_EDGES // NS
E_CHUNK = 4000
E_CHUNK2 = 2000
E2_PER_TILE = N_EDGES // NW
G2 = 80

R128 = N_EDGES * EDGE_DIM // 128
BLK128 = 1000
NBLK = R128 // BLK128


@functools.cache
def _mesh():
    return plsc.VectorSubcoreMesh(
        core_axis_name="c", subcore_axis_name="s",
        num_cores=NC, num_subcores=NS)


_SC_PARAMS = dict(
    compiler_params=pltpu.CompilerParams(
        use_tc_tiling_on_sc=False, needs_layout_passes=False))


def _proj_body(x_ref, wt_ref, out_ref):
    out_ref[...] = 0.5 * jnp.dot(
        x_ref[...], wt_ref[...], preferred_element_type=jnp.float32)


def _proj(x, w_t):
    return pl.pallas_call(
        _proj_body,
        out_shape=jax.ShapeDtypeStruct((N_NODES, EDGE_DIM), jnp.float32),
    )(x, w_t)


def _fused_body(xp_hbm, src_hbm, dst_hbm, ea_hbm, out_hbm,
                idx_s, idx_d, rows_s, rows_d, acc, sem_s, sem_d):
    c = lax.axis_index("c")
    s = lax.axis_index("s")
    wid = c * NS + s
    base = wid * E2_PER_TILE
    nchunks = E2_PER_TILE // E_CHUNK2

    def chunk(k, _):
        off = base + k * E_CHUNK2
        pltpu.sync_copy(src_hbm.at[pl.ds(off, E_CHUNK2)], idx_s)
        pltpu.sync_copy(dst_hbm.at[pl.ds(off, E_CHUNK2)], idx_d)
        pltpu.sync_copy(ea_hbm.at[pl.ds(off, E_CHUNK2)], acc)

        def batch(b, _):
            boff = pl.multiple_of(b * G2, 8)
            cp_s = pltpu.async_copy(
                xp_hbm.at[idx_s.at[pl.ds(boff, G2)]], rows_s, sem_s)
            cp_d = pltpu.async_copy(
                xp_hbm.at[idx_d.at[pl.ds(boff, G2)]], rows_d, sem_d)
            cp_s.wait()
            cp_d.wait()

            def row(r, _):
                acc[boff + r] = acc[boff + r] + rows_s[r] + rows_d[r]
                return 0

            lax.fori_loop(0, G2, row, 0)
            return 0

        lax.fori_loop(0, E_CHUNK2 // G2, batch, 0)
        pltpu.sync_copy(acc, out_hbm.at[pl.ds(off, E_CHUNK2)])
        return 0

    lax.fori_loop(0, nchunks, chunk, 0)


def _fused_stage(xp_half, src, dst, edge_attr):
    k = pl.kernel(
        _fused_body,
        out_type=jax.ShapeDtypeStruct((N_EDGES, EDGE_DIM), jnp.float32),
        mesh=_mesh(), **_SC_PARAMS,
        scratch_types=[
            pltpu.VMEM((E_CHUNK2,), jnp.int32),
            pltpu.VMEM((E_CHUNK2,), jnp.int32),
            pltpu.VMEM((G2, EDGE_DIM), jnp.float32),
            pltpu.VMEM((G2, EDGE_DIM), jnp.float32),
            pltpu.VMEM((E_CHUNK2, EDGE_DIM), jnp.float32),
            pltpu.SemaphoreType.DMA,
            pltpu.SemaphoreType.DMA,
        ],
    )
    return k(xp_half, src, dst, edge_attr)


def _filter_chunk(vals, keys, nvec, seg_base, seg_span, clist, dlist,
                  val_is_pos, off):
    def vec(i, cur):
        d = keys[pl.ds(i * 16, 16)]
        loc = d - seg_base
        mask = (loc >= 0) & (loc < seg_span)
        loc = jnp.where(mask, loc, seg_span)
        if val_is_pos:
            u = off + i * 16 + lax.iota(jnp.int32, 16)
        else:
            u = vals[pl.ds(i * 16, 16)]
        plsc.store_compressed(clist.at[pl.ds(cur, 16)], u, mask=mask)
        plsc.store_compressed(dlist.at[pl.ds(cur, 16)], loc, mask=mask)
        return cur + jnp.sum(mask.astype(jnp.int32))

    nc = lax.fori_loop(0, nvec, vec, 0)

    zi = jnp.zeros((16,), jnp.int32)
    di = jnp.full((16,), seg_span, jnp.int32)
    for j in range(2 * G // 16):
        clist[pl.ds(nc + j * 16, 16)] = zi
        dlist[pl.ds(nc + j * 16, 16)] = di
    return nc


def _drain_chunk(nc, table_hbm, clist, dlist, dbufA, dbufB, rowsA, rowsB,
                 ones, sums_sp, cnts_sp, semA, semB):
    nb2 = 2 * ((nc + 2 * G - 1) // (2 * G))

    def gather(b, rows, sem):
        boff = pl.multiple_of(b * G, 8)
        return pltpu.async_copy(
            table_hbm.at[clist.at[pl.ds(boff, G)]], rows, sem)

    def consume(b, rows, dbuf):
        boff = pl.multiple_of(b * G, 8)
        for j in range(G // 16):
            dbuf[pl.ds(j * 16, 16)] = dlist[pl.ds(boff + j * 16, 16)]
        pltpu.sync_copy(rows, sums_sp.at[dbuf], add=True)
        pltpu.sync_copy(ones, cnts_sp.at[dbuf], add=True)

    gather(0, rowsA, semA)

    def pair(ip, _):
        b0 = ip * 2
        gather(b0 + 1, rowsB, semB)
        pltpu.make_async_copy(
            table_hbm.at[pl.ds(0, G)], rowsA, semA).wait()
        consume(b0, rowsA, dbufA)

        @pl.when(b0 + 2 < nb2)
        def _():
            gather(b0 + 2, rowsA, semA)

        pltpu.make_async_copy(
            table_hbm.at[pl.ds(0, G)], rowsB, semB).wait()
        consume(b0 + 1, rowsB, dbufB)
        return 0

    lax.fori_loop(0, nb2 // 2, pair, 0)


def _zero_region(s, tile_rows, zb, zc, sums_sp, cnts_sp):
    row0 = s * tile_rows
    nfull = tile_rows // ZROWS
    for j in range(nfull):
        pltpu.sync_copy(zb, sums_sp.at[pl.ds(row0 + j * ZROWS, ZROWS)])
        pltpu.sync_copy(zc, cnts_sp.at[pl.ds(row0 + j * ZROWS, ZROWS)])
    rem = tile_rows - nfull * ZROWS
    if rem:
        pltpu.sync_copy(zb.at[pl.ds(0, rem)],
                        sums_sp.at[pl.ds(row0 + nfull * ZROWS, rem)])
        pltpu.sync_copy(zc.at[pl.ds(0, rem)],
                        cnts_sp.at[pl.ds(row0 + nfull * ZROWS, rem)])


def _divide_flush(s, tile_rows, out_base, dvb, cvb, sums_sp, cnts_sp,
                  out_hbm):
    row0 = s * tile_rows
    nfull = tile_rows // DCH

    def do(roff, n):
        pltpu.sync_copy(sums_sp.at[pl.ds(row0 + roff, n)], dvb.at[pl.ds(0, n)])
        pltpu.sync_copy(cnts_sp.at[pl.ds(row0 + roff, n)], cvb.at[pl.ds(0, n)])

        def row(r, _):
            cnt = plsc.load_gather(cvb, [lax.broadcast(r, (16,))])
            dvb[r] = dvb[r] / jnp.maximum(cnt, 1.0)
            return 0

        lax.fori_loop(0, n, row, 0)
        pltpu.sync_copy(dvb.at[pl.ds(0, n)],
                        out_hbm.at[pl.ds(out_base + row0 + roff, n)])

    for j in range(nfull):
        do(j * DCH, DCH)
    rem = tile_rows - nfull * DCH
    if rem:
        do(nfull * DCH, rem)


def _init_const(ones, zb, zc):
    one_v = jnp.ones((16,), jnp.float32)
    zero_v = jnp.zeros((16,), jnp.float32)
    for j in range(G // 16):
        ones[pl.ds(j * 16, 16)] = one_v
    for j in range(ZROWS // 16):
        zc[pl.ds(j * 16, 16)] = zero_v

    def zrow(r, _):
        zb[r] = zero_v
        return 0

    lax.fori_loop(0, ZROWS, zrow, 0)


def _line_body(fused_hbm, lsrc_hbm, ldst_hbm, agg_hbm,
               lsrc, ldst, clist, dlist, dbufA, dbufB, rowsA, rowsB,
               ones, zb, zc, dvb, cvb, sums_sp, cnts_sp, semA, semB):
    c = lax.axis_index("c")
    s = lax.axis_index("s")
    _init_const(ones, zb, zc)
    ibase = s * L_PER_TILE

    for p in range(NP):
        sid = c * NP + p
        seg_base = sid * SEG_S
        _zero_region(s, TILE_SEG, zb, zc, sums_sp, cnts_sp)
        plsc.subcore_barrier()

        def chunk(k, _):
            off = ibase + k * L_CHUNK
            pltpu.sync_copy(lsrc_hbm.at[pl.ds(off, L_CHUNK)], lsrc)
            pltpu.sync_copy(ldst_hbm.at[pl.ds(off, L_CHUNK)], ldst)
            nc = _filter_chunk(lsrc, ldst, L_CHUNK // 16, seg_base, SEG_S,
                               clist, dlist, False, 0)
            _drain_chunk(nc, fused_hbm, clist, dlist, dbufA, dbufB,
                         rowsA, rowsB, ones, sums_sp, cnts_sp, semA, semB)
            return 0

        lax.fori_loop(0, L_PER_TILE // L_CHUNK, chunk, 0)
        plsc.subcore_barrier()
        _divide_flush(s, TILE_SEG, seg_base, dvb, cvb, sums_sp, cnts_sp,
                      agg_hbm)
        plsc.subcore_barrier()


def _line_stage(fused, lsrc, ldst):
    k = pl.kernel(
        _line_body,
        out_type=jax.ShapeDtypeStruct((AGG_ROWS, EDGE_DIM), jnp.float32),
        mesh=_mesh(), **_SC_PARAMS,
        scratch_types=[
            pltpu.VMEM((L_CHUNK,), jnp.int32),
            pltpu.VMEM((L_CHUNK,), jnp.int32),
            pltpu.VMEM((CLIST,), jnp.int32),
            pltpu.VMEM((CLIST,), jnp.int32),
            pltpu.VMEM((G,), jnp.int32),
            pltpu.VMEM((G,), jnp.int32),
            pltpu.VMEM((G, EDGE_DIM), jnp.float32),
            pltpu.VMEM((G, EDGE_DIM), jnp.float32),
            pltpu.VMEM((G,), jnp.float32),
            pltpu.VMEM((ZROWS, EDGE_DIM), jnp.float32),
            pltpu.VMEM((ZROWS,), jnp.float32),
            pltpu.VMEM((DCH, EDGE_DIM), jnp.float32),
            pltpu.VMEM((DCH,), jnp.float32),
            pltpu.VMEM_SHARED((SEG_SP, EDGE_DIM), jnp.float32),
            pltpu.VMEM_SHARED((SEG_SP,), jnp.float32),
            pltpu.SemaphoreType.DMA,
            pltpu.SemaphoreType.DMA,
        ],
    )
    return k(fused, lsrc, ldst)


def _mlp_body(agg_ref, w_ref, b_ref, a_ref, pre_ref, stats_ref, acc_ref):
    i = pl.program_id(0)
    h = jnp.dot(agg_ref[...], w_ref[...], preferred_element_type=jnp.float32)
    h = h + b_ref[...]
    h = jnp.where(h >= 0.0, h, a_ref[0, 0] * h)
    pre_ref[...] = h

    @pl.when(i == 0)
    def _():
        acc_ref[...] = jnp.zeros_like(acc_ref)

    part = jnp.concatenate(
        [jnp.sum(h, axis=0, keepdims=True),
         jnp.sum(h * h, axis=0, keepdims=True)], axis=0)
    acc_ref[0:2, :] = acc_ref[0:2, :] + part

    @pl.when(i == NBLK - 1)
    def _():
        stats_ref[...] = acc_ref[...]


def _mlp_stage(agg128, w128, b128, ar):
    return pl.pallas_call(
        _mlp_body,
        grid=(NBLK,),
        in_specs=[
            pl.BlockSpec((BLK128, 128), lambda i: (i, 0)),
            pl.BlockSpec((128, 128), lambda i: (0, 0)),
            pl.BlockSpec((1, 128), lambda i: (0, 0)),
            pl.BlockSpec((1, 1), lambda i: (0, 0)),
        ],
        out_specs=[
            pl.BlockSpec((BLK128, 128), lambda i: (i, 0)),
            pl.BlockSpec((8, 128), lambda i: (0, 0)),
        ],
        out_shape=[
            jax.ShapeDtypeStruct((R128, 128), jnp.float32),
            jax.ShapeDtypeStruct((8, 128), jnp.float32),
        ],
        scratch_shapes=[pltpu.VMEM((8, 128), jnp.float32)],
    )(agg128, w128, b128, ar)


def _bn_body(pre_ref, fused_ref, sc_ref, sh_ref, out_ref):
    out_ref[...] = fused_ref[...] + pre_ref[...] * sc_ref[...] + sh_ref[...]


def _bn_stage(pre, fused128, scale128, shift128):
    return pl.pallas_call(
        _bn_body,
        grid=(NBLK,),
        in_specs=[
            pl.BlockSpec((BLK128, 128), lambda i: (i, 0)),
            pl.BlockSpec((BLK128, 128), lambda i: (i, 0)),
            pl.BlockSpec((1, 128), lambda i: (0, 0)),
            pl.BlockSpec((1, 128), lambda i: (0, 0)),
        ],
        out_specs=pl.BlockSpec((BLK128, 128), lambda i: (i, 0)),
        out_shape=jax.ShapeDtypeStruct((R128, 128), jnp.float32),
    )(pre, fused128, scale128, shift128)


def _node_body(f2_hbm, dst_hbm, out_hbm,
               didx, clist, dlist, dbufA, dbufB, rowsA, rowsB,
               ones, zb, zc, dvb, cvb, sums_sp, cnts_sp, semA, semB):
    c = lax.axis_index("c")
    s = lax.axis_index("s")
    _init_const(ones, zb, zc)
    seg_base = c * N_S
    ibase = s * E_PER_TILE

    _zero_region(s, NTILE_SEG, zb, zc, sums_sp, cnts_sp)
    plsc.subcore_barrier()

    def chunk(k, _):
        off = ibase + k * E_CHUNK
        pltpu.sync_copy(dst_hbm.at[pl.ds(off, E_CHUNK)], didx)
        nc = _filter_chunk(None, didx, E_CHUNK // 16, seg_base, N_S,
                           clist, dlist, True, off)
        _drain_chunk(nc, f2_hbm, clist, dlist, dbufA, dbufB,
                     rowsA, rowsB, ones, sums_sp, cnts_sp, semA, semB)
        return 0

    lax.fori_loop(0, E_PER_TILE // E_CHUNK, chunk, 0)
    plsc.subcore_barrier()
    _divide_flush(s, NTILE_SEG, seg_base, dvb, cvb, sums_sp, cnts_sp,
                  out_hbm)


def _node_stage(fused2, dst):
    k = pl.kernel(
        _node_body,
        out_type=jax.ShapeDtypeStruct((NC * N_S, EDGE_DIM), jnp.float32),
        mesh=_mesh(), **_SC_PARAMS,
        scratch_types=[
            pltpu.VMEM((E_CHUNK,), jnp.int32),
            pltpu.VMEM((CLIST,), jnp.int32),
            pltpu.VMEM((CLIST,), jnp.int32),
            pltpu.VMEM((G,), jnp.int32),
            pltpu.VMEM((G,), jnp.int32),
            pltpu.VMEM((G, EDGE_DIM), jnp.float32),
            pltpu.VMEM((G, EDGE_DIM), jnp.float32),
            pltpu.VMEM((G,), jnp.float32),
            pltpu.VMEM((ZROWS, EDGE_DIM), jnp.float32),
            pltpu.VMEM((ZROWS,), jnp.float32),
            pltpu.VMEM((DCH, EDGE_DIM), jnp.float32),
            pltpu.VMEM((DCH,), jnp.float32),
            pltpu.VMEM_SHARED((N_SP, EDGE_DIM), jnp.float32),
            pltpu.VMEM_SHARED((N_SP,), jnp.float32),
            pltpu.SemaphoreType.DMA,
            pltpu.SemaphoreType.DMA,
        ],
    )
    return k(fused2, dst)


def kernel(x, edge_index, edge_attr, line_graph_edge_index,
           W_proj, W1, b1, prelu_a, bn_gamma, bn_beta):
    src = edge_index[0]
    dst = edge_index[1]
    xp_half = _proj(x, W_proj.T)
    fused = _fused_stage(xp_half, src, dst, edge_attr)
    agg = _line_stage(
        fused, line_graph_edge_index[0], line_graph_edge_index[1])

    agg128 = agg.reshape(AGG_ROWS * EDGE_DIM // 128, 128)
    w128 = jax.scipy.linalg.block_diag(*([W1.T] * 8))
    b128 = jnp.tile(b1, 8).reshape(1, 128)
    pre, stats = _mlp_stage(agg128, w128, b128, prelu_a.reshape(1, 1))

    n = float(N_EDGES)
    mu = stats[0].reshape(8, EDGE_DIM).sum(axis=0) / n
    var = stats[1].reshape(8, EDGE_DIM).sum(axis=0) / n - mu * mu
    inv = lax.rsqrt(var + 1e-5)
    scale = inv * bn_gamma
    shift = bn_beta - mu * scale
    scale128 = jnp.tile(scale, 8).reshape(1, 128)
    shift128 = jnp.tile(shift, 8).reshape(1, 128)

    fused128 = fused.reshape(R128, 128)
    fused2 = _bn_stage(pre, fused128, scale128, shift128)
    out = _node_stage(fused2.reshape(N_EDGES, EDGE_DIM), dst)
    return out[:N_NODES]
